# baseline (speedup 1.0000x reference)
import functools

import jax
import jax.numpy as jnp
from jax import lax
from jax.experimental import pallas as pl
from jax.experimental.pallas import tpu as pltpu
from jax.experimental.pallas import tpu_sc as plsc

EMBED = 64
NUM_CORES = 2
NUM_SUBCORES = 16
NW = NUM_CORES * NUM_SUBCORES
CB = 256


def _detile_kernel(vocab, table_t, tail_t, out_a,
                   in0, in1, col0, col1, si0, si1, so0, so1):
    nblk = vocab // 256
    per_w = nblk // NW
    rem = nblk - per_w * NW
    n_pairs = per_w // 2
    wid = lax.axis_index("s") * NUM_CORES + lax.axis_index("c")

    def in_copy(i, buf, sem):
        b = wid + i * NW
        return pltpu.make_async_copy(
            table_t.at[:, pl.ds(b * 256, 256)], buf, sem)

    def o_copy(i, col, sem):
        b = wid + i * NW
        r0 = pl.multiple_of(b * 128, 8)
        return pltpu.make_async_copy(
            col.at[:, pl.ds(0, 128)], out_a.at[pl.ds(r0, 128), :], sem)

    def o_start(i, col, sem):
        o_copy(i, col, sem).start()

    def o_wait(i, col, sem):
        o_copy(i, col, sem).wait()

    lane = lax.iota(jnp.int32, 16)
    qvs = [8 * g + lane // 2 for g in range(16)]
    sv64 = lax.rem(lane, 2) * EMBED

    def transpose_a(buf, col, ngroups):
        @plsc.parallel_loop(0, EMBED, step=1, unroll=4)
        def _(e):
            cv = sv64 + e
            for g in range(ngroups):
                v = buf[e, pl.ds(16 * g, 16)]
                plsc.store_scatter(col, [qvs[g], cv], v)

    in_copy(0, in0, si0).start()

    def pair(p, _):
        i0 = p * 2
        i1 = i0 + 1
        in_copy(i1, in1, si1).start()
        in_copy(i0, in0, si0).wait()

        @pl.when(p > 0)
        def _():
            o_wait(i0, col0, so0)
        transpose_a(in0, col0, 16)
        o_start(i0, col0, so0)

        @pl.when(p + 1 < n_pairs)
        def _():
            in_copy(i0 + 2, in0, si0).start()

        in_copy(i1, in1, si1).wait()

        @pl.when(p > 0)
        def _():
            o_wait(i1, col1, so1)
        transpose_a(in1, col1, 16)
        o_start(i1, col1, so1)
        return ()

    lax.fori_loop(0, n_pairs, pair, (), unroll=False)
    o_wait(per_w - 2, col0, so0)
    o_wait(per_w - 1, col1, so1)

    @pl.when(wid < rem)
    def _():
        in_copy(per_w, in0, si0).start()
        in_copy(per_w, in0, si0).wait()
        transpose_a(in0, col0, 16)
        o_start(per_w, col0, so0)
        o_wait(per_w, col0, so0)

    tail = vocab - nblk * 256
    if tail:
        @pl.when(wid == rem)
        def _():
            cp = pltpu.make_async_copy(
                tail_t, in1.at[:, pl.ds(0, tail_t.shape[1])], si1)
            cp.start()
            cp.wait()
            transpose_a(in1, col1, tail // 16)
            r0 = pl.multiple_of(nblk * 128, 8)
            cpo = pltpu.make_async_copy(
                col1.at[pl.ds(0, tail // 2), pl.ds(0, 128)],
                out_a.at[pl.ds(r0, tail // 2), :], so1)
            cpo.start()
            cpo.wait()


def _emb_kernel(batch, hist, table_hbm, idx_hbm, out_hbm,
                idx_v, rows0, rows1, col0, col1, sl, sg0, sg1, ss0, ss1):
    b_blk = batch // NW
    halves = b_blk // CB
    n_chunks = hist * halves
    n_pairs = n_chunks // 2
    wid = lax.axis_index("s") * NUM_CORES + lax.axis_index("c")
    b0 = pl.multiple_of(wid * b_blk, 8)

    pltpu.async_copy(idx_hbm.at[:, pl.ds(b0, b_blk)], idx_v, sl).wait()

    def g_copy(c, rows, sem):
        h = c // halves
        off = pl.multiple_of((c % halves) * CB, 8)
        return pltpu.make_async_copy(
            table_hbm.at[idx_v.at[h, pl.ds(off, CB)]], rows, sem)

    def s_copies(c, col, sem):
        h = c // halves
        off = (c % halves) * CB
        bt_abs = (b0 + off) // 128
        return [
            pltpu.make_async_copy(
                col.at[btp, :, :, pl.ds(0, 128)],
                out_hbm.at[h, :, bt_abs + btp, :, :], sem)
            for btp in range(CB // 128)
        ]

    def s_start(c, col, sem):
        for cp in s_copies(c, col, sem):
            cp.start()

    def s_wait(c, col, sem):
        for cp in s_copies(c, col, sem):
            cp.wait()

    lane = lax.iota(jnp.int32, 16)
    ei_v = lax.rem(lane, 8)
    et_vs = [lane // 8 + 2 * k for k in range(EMBED // 16)]

    def transpose(rows, col):
        @plsc.parallel_loop(0, CB, step=1, unroll=2)
        def _(j):
            bt_v = jnp.full((16,), j // 128, dtype=jnp.int32)
            bi_v = jnp.full((16,), j % 128, dtype=jnp.int32)
            for k in range(EMBED // 16):
                v = rows[j, pl.ds(16 * k, 16)]
                plsc.store_scatter(col, [bt_v, et_vs[k], ei_v, bi_v], v)

    g_copy(0, rows0, sg0).start()

    def pair(p, _):
        c0 = p * 2
        c1 = c0 + 1
        g_copy(c1, rows1, sg1).start()
        g_copy(c0, rows0, sg0).wait()

        @pl.when(p > 0)
        def _():
            s_wait(c0, col0, ss0)
        transpose(rows0, col0)
        s_start(c0, col0, ss0)

        @pl.when(p + 1 < n_pairs)
        def _():
            g_copy(c0 + 2, rows0, sg0).start()

        g_copy(c1, rows1, sg1).wait()

        @pl.when(p > 0)
        def _():
            s_wait(c1, col1, ss1)
        transpose(rows1, col1)
        s_start(c1, col1, ss1)
        return ()

    lax.fori_loop(0, n_pairs, pair, (), unroll=False)
    s_wait(n_chunks - 2, col0, ss0)
    s_wait(n_chunks - 1, col1, ss1)


def kernel(indices, table):
    batch, hist = indices.shape
    vocab = table.shape[0]
    idx_t = indices.T.astype(jnp.int32)

    mesh = plsc.VectorSubcoreMesh(
        core_axis_name="c", subcore_axis_name="s",
        num_cores=NUM_CORES, num_subcores=NUM_SUBCORES,
    )

    detile = pl.kernel(
        functools.partial(_detile_kernel, vocab),
        out_type=jax.ShapeDtypeStruct((vocab // 2, 2 * EMBED), jnp.float32),
        mesh=mesh,
        scratch_types=[
            pltpu.VMEM((EMBED, 256), jnp.float32),
            pltpu.VMEM((EMBED, 256), jnp.float32),
            pltpu.VMEM((128, 129), jnp.float32),
            pltpu.VMEM((128, 129), jnp.float32),
            pltpu.SemaphoreType.DMA,
            pltpu.SemaphoreType.DMA,
            pltpu.SemaphoreType.DMA,
            pltpu.SemaphoreType.DMA,
        ],
        compiler_params=pltpu.CompilerParams(
            use_tc_tiling_on_sc=True, needs_layout_passes=False),
    )
    nblk = vocab // 256
    tail_t = jnp.pad(table[nblk * 256:].T,
                     ((0, 0), (0, (-vocab) % 128)))
    table_rm = detile(table.T, tail_t).reshape(vocab, EMBED)

    gather = pl.kernel(
        functools.partial(_emb_kernel, batch, hist),
        out_type=jax.ShapeDtypeStruct(
            (hist, EMBED // 8, batch // 128, 8, 128), jnp.float32),
        mesh=mesh,
        scratch_types=[
            pltpu.VMEM((hist, batch // NW), jnp.int32),
            pltpu.VMEM((CB, EMBED), jnp.float32),
            pltpu.VMEM((CB, EMBED), jnp.float32),
            pltpu.VMEM((CB // 128, EMBED // 8, 8, 129), jnp.float32),
            pltpu.VMEM((CB // 128, EMBED // 8, 8, 129), jnp.float32),
            pltpu.SemaphoreType.DMA,
            pltpu.SemaphoreType.DMA,
            pltpu.SemaphoreType.DMA,
            pltpu.SemaphoreType.DMA,
            pltpu.SemaphoreType.DMA,
        ],
        compiler_params=pltpu.CompilerParams(
            use_tc_tiling_on_sc=False, needs_layout_passes=False),
    )
    out5 = gather(table_rm, idx_t)
    out3 = out5.transpose(0, 1, 3, 2, 4).reshape(hist, EMBED, batch)
    return out3.transpose(2, 0, 1)


# speedup vs baseline: 1.3689x; 1.3689x over previous
import functools

import jax
import jax.numpy as jnp
from jax import lax
from jax.experimental import pallas as pl
from jax.experimental.pallas import tpu as pltpu
from jax.experimental.pallas import tpu_sc as plsc

EMBED = 64
NUM_CORES = 2
NUM_SUBCORES = 16
NW = NUM_CORES * NUM_SUBCORES
CB = 256
CP = CB + 1


def _emb_kernel(batch, hist, table_hbm, idx_hbm, out_hbm,
                idx_v, rows0, rows1, col0, col1, sl, sg0, sg1, ss0, ss1):
    b_blk = batch // NW
    halves = b_blk // CB
    n_chunks = hist * halves
    n_pairs = n_chunks // 2
    wid = lax.axis_index("s") * NUM_CORES + lax.axis_index("c")
    b0 = pl.multiple_of(wid * b_blk, 8)

    pltpu.async_copy(idx_hbm.at[:, pl.ds(b0, b_blk)], idx_v, sl).wait()

    def g_copy(c, rows, sem):
        h = c // halves
        off = pl.multiple_of((c % halves) * CB, 8)
        return pltpu.make_async_copy(
            table_hbm.at[idx_v.at[h, pl.ds(off, CB)]], rows, sem)

    def s_copies(c, col, sem):
        h = c // halves
        off = (c % halves) * CB
        bt_abs = (b0 + off) // 128
        return [
            pltpu.make_async_copy(
                col.at[btp, :, :, pl.ds(0, 128)],
                out_hbm.at[h, :, bt_abs + btp, :, :], sem)
            for btp in range(CB // 128)
        ]

    def s_start(c, col, sem):
        for cp in s_copies(c, col, sem):
            cp.start()

    def s_wait(c, col, sem):
        for cp in s_copies(c, col, sem):
            cp.wait()

    lane = lax.iota(jnp.int32, 16)
    ei_v = lax.rem(lane, 8)
    et_vs = [lane // 8 + 2 * k for k in range(EMBED // 16)]

    def transpose(rows, col):
        @plsc.parallel_loop(0, CB, step=1, unroll=2)
        def _(j):
            bt_v = jnp.full((16,), j // 128, dtype=jnp.int32)
            bi_v = jnp.full((16,), j % 128, dtype=jnp.int32)
            for k in range(EMBED // 16):
                v = rows[j, pl.ds(16 * k, 16)]
                plsc.store_scatter(col, [bt_v, et_vs[k], ei_v, bi_v], v)

    g_copy(0, rows0, sg0).start()

    def pair(p, _):
        c0 = p * 2
        c1 = c0 + 1
        g_copy(c1, rows1, sg1).start()
        g_copy(c0, rows0, sg0).wait()

        @pl.when(p > 0)
        def _():
            s_wait(c0, col0, ss0)
        transpose(rows0, col0)
        s_start(c0, col0, ss0)

        @pl.when(p + 1 < n_pairs)
        def _():
            g_copy(c0 + 2, rows0, sg0).start()

        g_copy(c1, rows1, sg1).wait()

        @pl.when(p > 0)
        def _():
            s_wait(c1, col1, ss1)
        transpose(rows1, col1)
        s_start(c1, col1, ss1)
        return ()

    lax.fori_loop(0, n_pairs, pair, (), unroll=False)
    s_wait(n_chunks - 2, col0, ss0)
    s_wait(n_chunks - 1, col1, ss1)


def kernel(indices, table):
    batch, hist = indices.shape
    idx_t = indices.T.astype(jnp.int32)

    mesh = plsc.VectorSubcoreMesh(
        core_axis_name="c", subcore_axis_name="s",
        num_cores=NUM_CORES, num_subcores=NUM_SUBCORES,
    )
    k = pl.kernel(
        functools.partial(_emb_kernel, batch, hist),
        out_type=jax.ShapeDtypeStruct(
            (hist, EMBED // 8, batch // 128, 8, 128), jnp.float32),
        mesh=mesh,
        scratch_types=[
            pltpu.VMEM((hist, batch // NW), jnp.int32),
            pltpu.VMEM((CB, EMBED), jnp.float32),
            pltpu.VMEM((CB, EMBED), jnp.float32),
            pltpu.VMEM((CB // 128, EMBED // 8, 8, 129), jnp.float32),
            pltpu.VMEM((CB // 128, EMBED // 8, 8, 129), jnp.float32),
            pltpu.SemaphoreType.DMA,
            pltpu.SemaphoreType.DMA,
            pltpu.SemaphoreType.DMA,
            pltpu.SemaphoreType.DMA,
            pltpu.SemaphoreType.DMA,
        ],
        compiler_params=pltpu.CompilerParams(
            use_tc_tiling_on_sc=False, needs_layout_passes=False),
    )
    out5 = k(table, idx_t)
    out3 = out5.transpose(0, 1, 3, 2, 4).reshape(hist, EMBED, batch)
    return out3.transpose(2, 0, 1)
